# Initial kernel scaffold; baseline (speedup 1.0000x reference)
import jax
import jax.numpy as jnp
from jax.experimental import pallas as pl


def kernel(x, edge_index, edge_attr, batch, W_in, b_in, We, be, W1, b1, gm, bm, W2, b2, go, bo, eps, Wh1, bh1, gh, bh, Wh2, bh2):
    raise NotImplementedError("write your pallas kernel here")


# speedup vs baseline: 1.8241x; 1.8241x over previous
import functools

import jax
import jax.numpy as jnp
from jax import lax
from jax.experimental import pallas as pl
from jax.experimental.pallas import tpu as pltpu
from jax.experimental.pallas import tpu_sc as plsc

N = 10000
E = 320000
D = 128
H = 128
ED = 4
L = 3
G = 64
BN_EPS = 1e-5

NC = 2
NS = 16
NW = NC * NS
EPW = E // NW
C = 80
NCH = EPW // C
RPS = N // NS
ZR = 125
VL = 16
PCP = 16

@functools.cache
def _mesh():
    return plsc.VectorSubcoreMesh(
        core_axis_name="c", subcore_axis_name="s",
        num_cores=NC, num_subcores=NS)


_SC_PARAMS = pltpu.CompilerParams(needs_layout_passes=False)


def _lin_body(x_ref, w_ref, b_ref, o_ref):
    o_ref[...] = (
        jnp.dot(x_ref[...], w_ref[...], preferred_element_type=jnp.float32)
        + b_ref[...])


def _linear(x, w, b, bm=1000):
    n, d = x.shape
    h = w.shape[1]
    return pl.pallas_call(
        _lin_body,
        grid=(n // bm,),
        in_specs=[
            pl.BlockSpec((bm, d), lambda i: (i, 0)),
            pl.BlockSpec((d, h), lambda i: (0, 0)),
            pl.BlockSpec((1, h), lambda i: (0, 0)),
        ],
        out_specs=pl.BlockSpec((bm, h), lambda i: (i, 0)),
        out_shape=jax.ShapeDtypeStruct((n, h), jnp.float32),
    )(x, w, b.reshape(1, h))


def _emat_body(ea_ref, we_ref, be_ref, o_ref):
    ea = ea_ref[...]
    w = we_ref[0]
    e = be_ref[0]
    for k in range(ED):
        e = e + ea[:, k:k + 1] * w[k:k + 1, :]
    o_ref[0] = e


def _emat(edge_attr, We, be, bm=2000):
    return pl.pallas_call(
        _emat_body,
        grid=(L, E // bm),
        in_specs=[
            pl.BlockSpec((bm, ED), lambda l, i: (i, 0)),
            pl.BlockSpec((1, ED, H), lambda l, i: (l, 0, 0)),
            pl.BlockSpec((1, 1, H), lambda l, i: (l, 0, 0)),
        ],
        out_specs=pl.BlockSpec((1, bm, H), lambda l, i: (l, i, 0)),
        out_shape=jax.ShapeDtypeStruct((L, E, H), jnp.float32),
    )(edge_attr, We, be.reshape(L, 1, H))


def _mlp_body(h_ref, p_ref, sc_ref, w1_ref, b1_ref, w2_ref, b2_ref, o_ref):
    z = sc_ref[...] * h_ref[...] + p_ref[0] + p_ref[1]
    z = jnp.dot(z, w1_ref[...], preferred_element_type=jnp.float32) + b1_ref[...]
    z = jnp.maximum(z, 0.0)
    z = jnp.dot(z, w2_ref[...], preferred_element_type=jnp.float32) + b2_ref[...]
    o_ref[...] = jnp.maximum(z, 0.0)


def _mlp(h, parts, scale_row, w1, b1, w2, b2, bm=1000):
    return pl.pallas_call(
        _mlp_body,
        grid=(N // bm,),
        in_specs=[
            pl.BlockSpec((bm, H), lambda i: (i, 0)),
            pl.BlockSpec((NC, bm, H), lambda i: (0, i, 0)),
            pl.BlockSpec((1, H), lambda i: (0, 0)),
            pl.BlockSpec((H, H), lambda i: (0, 0)),
            pl.BlockSpec((1, H), lambda i: (0, 0)),
            pl.BlockSpec((H, H), lambda i: (0, 0)),
            pl.BlockSpec((1, H), lambda i: (0, 0)),
        ],
        out_specs=pl.BlockSpec((bm, H), lambda i: (i, 0)),
        out_shape=jax.ShapeDtypeStruct((N, H), jnp.float32),
    )(h, parts, scale_row, w1, b1.reshape(1, H), w2, b2.reshape(1, H))


def _head_body(mn_ref, mx_ref, a1_ref, a2_ref, b1_ref, w2_ref, b2_ref, o_ref):
    t = (jnp.dot(mn_ref[...], a1_ref[...], preferred_element_type=jnp.float32)
         + jnp.dot(mx_ref[...], a2_ref[...], preferred_element_type=jnp.float32)
         + b1_ref[...])
    t = jnp.maximum(t, 0.0)
    o_ref[...] = (
        jnp.dot(t, w2_ref[...], preferred_element_type=jnp.float32) + b2_ref[...])


def _head(means, maxes, a1, a2, b1, w2, b2):
    return pl.pallas_call(
        _head_body,
        out_shape=jax.ShapeDtypeStruct((G, 1), jnp.float32),
    )(means, maxes, a1, a2, b1.reshape(1, H), w2, b2.reshape(1, 1))


@functools.cache
def _edge_pass_kernel():
    return pl.kernel(
        _edge_pass_body,
        out_type=jax.ShapeDtypeStruct((NC * N, H), jnp.float32),
        mesh=_mesh(),
        scratch_types=[
            pltpu.VMEM((C,), jnp.int32),
            pltpu.VMEM((C,), jnp.int32),
            pltpu.VMEM((C, H), jnp.float32),
            pltpu.VMEM((C, H), jnp.float32),
            pltpu.VMEM((C, H), jnp.float32),
            pltpu.VMEM_SHARED((N, H), jnp.float32),
            pltpu.SemaphoreType.DMA,
        ],
        compiler_params=_SC_PARAMS,
    )


def _edge_pass_body(src_hbm, dst_hbm, em_hbm, h_hbm, out_hbm,
                    idx_s, idx_d, rows, ev, zbuf, aggr, sem):
    c = lax.axis_index("c")
    s = lax.axis_index("s")
    w = c * NS + s

    def zrow(i, carry):
        for j in range(H // VL):
            zbuf[i, pl.ds(VL * j, VL)] = jnp.zeros((VL,), jnp.float32)
        return carry

    lax.fori_loop(0, C, zrow, 0)
    for t in range((N // C + NS - 1) // NS):
        i = s + NS * t
        @pl.when(i < N // C)
        def _():
            pltpu.sync_copy(zbuf, aggr.at[pl.ds(i * C, C), :])
    plsc.subcore_barrier()

    base = pl.multiple_of(w * EPW, 8)

    def chunk(k, carry):
        eb = pl.multiple_of(base + k * C, 8)
        pltpu.sync_copy(src_hbm.at[pl.ds(eb, C)], idx_s)
        pltpu.sync_copy(dst_hbm.at[pl.ds(eb, C)], idx_d)
        pltpu.async_copy(h_hbm.at[idx_s], rows, sem).wait()
        pltpu.sync_copy(em_hbm.at[pl.ds(eb, C), :], ev)

        def edge(i, inner_carry):
            for j in range(H // VL):
                sl = pl.ds(VL * j, VL)
                rows[i, sl] = jnp.maximum(rows[i, sl] + ev[i, sl], 0.0)
            return inner_carry

        lax.fori_loop(0, C, edge, 0)
        pltpu.sync_copy(rows, aggr.at[idx_d], add=True)
        return carry

    lax.fori_loop(0, NCH, chunk, 0)
    plsc.subcore_barrier()
    for t in range((N // C + NS - 1) // NS):
        i = s + NS * t
        @pl.when(i < N // C)
        def _():
            pltpu.sync_copy(aggr.at[pl.ds(i * C, C), :],
                            out_hbm.at[pl.ds(c * N + i * C, C), :])


@functools.cache
def _pool_kernel():
    return pl.kernel(
        _pool_body,
        out_type=[jax.ShapeDtypeStruct((NW, 2, L * H), jnp.float32),
                  jax.ShapeDtypeStruct((NW, 2, L * H), jnp.float32)],
        mesh=_mesh(),
        scratch_types=[
            pltpu.VMEM((N,), jnp.int32),
            pltpu.VMEM((PCP, H), jnp.float32),
            pltpu.VMEM((PCP, H), jnp.float32),
            pltpu.VMEM((PCP, H), jnp.float32),
            pltpu.VMEM((L * H,), jnp.float32),
            pltpu.VMEM((L * H,), jnp.float32),
            pltpu.VMEM((2, L * H), jnp.float32),
            pltpu.VMEM((2, L * H), jnp.float32),
        ],
        compiler_params=_SC_PARAMS,
    )


def _pool_body(h1_hbm, h2_hbm, h3_hbm, b_hbm, mean_hbm, max_hbm,
               bv, r1, r2, r3, sacc, macc, mrow, xrow):
    c = lax.axis_index("c")
    s = lax.axis_index("s")
    w = c * NS + s
    pltpu.sync_copy(b_hbm, bv)
    hs = (h1_hbm, h2_hbm, h3_hbm)
    rs = (r1, r2, r3)

    for gl in range(2):
        g = 2 * w + gl

        def cnt(i, carry):
            sg, eg = carry
            v = bv[pl.ds(VL * i, VL)]
            sg = sg + jnp.sum((v < g).astype(jnp.int32))
            eg = eg + jnp.sum((v <= g).astype(jnp.int32))
            return sg, eg

        sg, eg = lax.fori_loop(0, N // VL, cnt,
                               (jnp.int32(0), jnp.int32(0)))

        for j in range(L * H // VL):
            sacc[pl.ds(VL * j, VL)] = jnp.zeros((VL,), jnp.float32)
            macc[pl.ds(VL * j, VL)] = jnp.full((VL,), -jnp.inf, jnp.float32)

        k0 = sg // PCP
        k1 = (eg + PCP - 1) // PCP

        def chunk(k, carry):
            rbase = k * PCP
            for t in range(L):
                pltpu.sync_copy(hs[t].at[pl.ds(rbase, PCP), :], rs[t])

            def row(i, inner_carry):
                ridx = jnp.full((VL,), rbase + i, jnp.int32)
                valid = (ridx >= sg) & (ridx < eg)
                for t in range(L):
                    for j in range(H // VL):
                        v = rs[t][i, pl.ds(VL * j, VL)]
                        asl = pl.ds(t * H + VL * j, VL)
                        sacc[asl] = sacc[asl] + jnp.where(valid, v, 0.0)
                        macc[asl] = jnp.maximum(
                            macc[asl], jnp.where(valid, v, -jnp.inf))
                return inner_carry

            lax.fori_loop(0, PCP, row, 0)
            return carry

        lax.fori_loop(k0, k1, chunk, 0)
        denom = jnp.maximum((eg - sg).astype(jnp.float32), 1.0)
        for j in range(L * H // VL):
            sl = pl.ds(VL * j, VL)
            mrow[gl, sl] = sacc[sl] / denom
            xrow[gl, sl] = macc[sl]

    pltpu.sync_copy(mrow, mean_hbm.at[w])
    pltpu.sync_copy(xrow, max_hbm.at[w])


def kernel(x, edge_index, edge_attr, batch, W_in, b_in, We, be, W1, b1,
           gm, bm, W2, b2, go, bo, eps, Wh1, bh1, gh, bh, Wh2, bh2):
    src = edge_index[0].astype(jnp.int32)
    dst = edge_index[1].astype(jnp.int32)
    batch32 = batch.astype(jnp.int32)

    s1 = gm / jnp.sqrt(1.0 + BN_EPS)
    w1f = W1 * s1[:, None, :]
    b1f = b1 * s1 + bm
    s2 = go / jnp.sqrt(1.0 + BN_EPS)
    w2f = W2 * s2[:, None, :]
    b2f = b2 * s2 + bo
    sh = gh / jnp.sqrt(1.0 + BN_EPS)
    wh1f = Wh1 * sh[None, :]
    bh1f = bh1 * sh + bh

    h = _linear(x, W_in, b_in)
    emat = _emat(edge_attr, We, be)

    outs = []
    for l in range(L):
        parts = _edge_pass_kernel()(src, dst, emat[l], h)
        scale_row = jnp.broadcast_to(
            (1.0 + eps[l]).astype(jnp.float32)[None, None], (1, H))
        h = _mlp(h, parts.reshape(NC, N, H), scale_row,
                 w1f[l], b1f[l], w2f[l], b2f[l])
        outs.append(h)

    means, maxes = _pool_kernel()(outs[0], outs[1], outs[2], batch32)
    o = _head(means.reshape(G, L * H), maxes.reshape(G, L * H),
              wh1f[:L * H], wh1f[L * H:], bh1f, Wh2, bh2)
    return o[:, 0]


# speedup vs baseline: 2.4736x; 1.3561x over previous
import functools

import jax
import jax.numpy as jnp
from jax import lax
from jax.experimental import pallas as pl
from jax.experimental.pallas import tpu as pltpu
from jax.experimental.pallas import tpu_sc as plsc

N = 10000
E = 320000
D = 128
H = 128
ED = 4
L = 3
G = 64
BN_EPS = 1e-5

NC = 2
NS = 16
NW = NC * NS
EPW = E // NW
C = 80
NCH = EPW // C
RPS = N // NS
ZR = 125
VL = 16
PCP = 16

@functools.cache
def _mesh():
    return plsc.VectorSubcoreMesh(
        core_axis_name="c", subcore_axis_name="s",
        num_cores=NC, num_subcores=NS)


_SC_PARAMS = pltpu.CompilerParams(needs_layout_passes=False)


def _lin_body(x_ref, w_ref, b_ref, o_ref):
    o_ref[...] = (
        jnp.dot(x_ref[...], w_ref[...], preferred_element_type=jnp.float32)
        + b_ref[...])


def _linear(x, w, b, bm=1000):
    n, d = x.shape
    h = w.shape[1]
    return pl.pallas_call(
        _lin_body,
        grid=(n // bm,),
        in_specs=[
            pl.BlockSpec((bm, d), lambda i: (i, 0)),
            pl.BlockSpec((d, h), lambda i: (0, 0)),
            pl.BlockSpec((1, h), lambda i: (0, 0)),
        ],
        out_specs=pl.BlockSpec((bm, h), lambda i: (i, 0)),
        out_shape=jax.ShapeDtypeStruct((n, h), jnp.float32),
    )(x, w, b.reshape(1, h))


def _mlp_body(h_ref, p_ref, sc_ref, w1_ref, b1_ref, w2_ref, b2_ref, o_ref):
    z = sc_ref[...] * h_ref[...] + p_ref[0] + p_ref[1]
    z = jnp.dot(z, w1_ref[...], preferred_element_type=jnp.float32) + b1_ref[...]
    z = jnp.maximum(z, 0.0)
    z = jnp.dot(z, w2_ref[...], preferred_element_type=jnp.float32) + b2_ref[...]
    o_ref[...] = jnp.maximum(z, 0.0)


def _mlp(h, parts, scale_row, w1, b1, w2, b2, bm=1000):
    return pl.pallas_call(
        _mlp_body,
        grid=(N // bm,),
        in_specs=[
            pl.BlockSpec((bm, H), lambda i: (i, 0)),
            pl.BlockSpec((NC, bm, H), lambda i: (0, i, 0)),
            pl.BlockSpec((1, H), lambda i: (0, 0)),
            pl.BlockSpec((H, H), lambda i: (0, 0)),
            pl.BlockSpec((1, H), lambda i: (0, 0)),
            pl.BlockSpec((H, H), lambda i: (0, 0)),
            pl.BlockSpec((1, H), lambda i: (0, 0)),
        ],
        out_specs=pl.BlockSpec((bm, H), lambda i: (i, 0)),
        out_shape=jax.ShapeDtypeStruct((N, H), jnp.float32),
    )(h, parts, scale_row, w1, b1.reshape(1, H), w2, b2.reshape(1, H))


def _head_body(mn_ref, mx_ref, a1_ref, a2_ref, b1_ref, w2_ref, b2_ref, o_ref):
    t = (jnp.dot(mn_ref[...], a1_ref[...], preferred_element_type=jnp.float32)
         + jnp.dot(mx_ref[...], a2_ref[...], preferred_element_type=jnp.float32)
         + b1_ref[...])
    t = jnp.maximum(t, 0.0)
    o_ref[...] = (
        jnp.dot(t, w2_ref[...], preferred_element_type=jnp.float32) + b2_ref[...])


def _head(means, maxes, a1, a2, b1, w2, b2):
    return pl.pallas_call(
        _head_body,
        out_shape=jax.ShapeDtypeStruct((G, 1), jnp.float32),
    )(means, maxes, a1, a2, b1.reshape(1, H), w2, b2.reshape(1, 1))


@functools.cache
def _edge_pass_kernel():
    return pl.kernel(
        _edge_pass_body,
        out_type=jax.ShapeDtypeStruct((NC * N, H), jnp.float32),
        mesh=_mesh(),
        scratch_types=[
            pltpu.VMEM((C,), jnp.int32),
            pltpu.VMEM((C,), jnp.int32),
            pltpu.VMEM((C, H), jnp.float32),
            pltpu.VMEM((C // 4, VL), jnp.float32),
            pltpu.VMEM((ED, H), jnp.float32),
            pltpu.VMEM((H,), jnp.float32),
            pltpu.VMEM((C, H), jnp.float32),
            pltpu.VMEM_SHARED((N, H), jnp.float32),
            pltpu.SemaphoreType.DMA,
        ],
        compiler_params=_SC_PARAMS,
    )


def _edge_pass_body(src_hbm, dst_hbm, ea_hbm, we_hbm, be_hbm, h_hbm, out_hbm,
                    idx_s, idx_d, rows, eav, wev, bev, zbuf, aggr, sem):
    c = lax.axis_index("c")
    s = lax.axis_index("s")
    w = c * NS + s
    pltpu.sync_copy(we_hbm, wev)
    pltpu.sync_copy(be_hbm, bev)

    def zrow(i, carry):
        for j in range(H // VL):
            zbuf[i, pl.ds(VL * j, VL)] = jnp.zeros((VL,), jnp.float32)
        return carry

    lax.fori_loop(0, C, zrow, 0)
    for t in range((N // C + NS - 1) // NS):
        i = s + NS * t
        @pl.when(i < N // C)
        def _():
            pltpu.sync_copy(zbuf, aggr.at[pl.ds(i * C, C), :])
    plsc.subcore_barrier()

    base = pl.multiple_of(w * EPW, 8)
    wk = [[wev[k, pl.ds(VL * j, VL)] for j in range(H // VL)]
          for k in range(ED)]
    bb = [bev[pl.ds(VL * j, VL)] for j in range(H // VL)]

    def chunk(k, carry):
        eb = pl.multiple_of(base + k * C, 8)
        pltpu.sync_copy(src_hbm.at[pl.ds(eb, C)], idx_s)
        pltpu.sync_copy(dst_hbm.at[pl.ds(eb, C)], idx_d)
        pltpu.async_copy(h_hbm.at[idx_s], rows, sem).wait()
        pltpu.sync_copy(ea_hbm.at[w * NCH + k], eav)

        def quad(q, inner_carry):
            av = eav[q, :]
            for je in range(4):
                i = 4 * q + je
                a = [av[4 * je + kk] for kk in range(ED)]
                for j in range(H // VL):
                    sl = pl.ds(VL * j, VL)
                    e = bb[j]
                    for kk in range(ED):
                        e = e + a[kk] * wk[kk][j]
                    rows[i, sl] = jnp.maximum(rows[i, sl] + e, 0.0)
            return inner_carry

        lax.fori_loop(0, C // 4, quad, 0)
        pltpu.sync_copy(rows, aggr.at[idx_d], add=True)
        return carry

    lax.fori_loop(0, NCH, chunk, 0)
    plsc.subcore_barrier()
    for t in range((N // C + NS - 1) // NS):
        i = s + NS * t
        @pl.when(i < N // C)
        def _():
            pltpu.sync_copy(aggr.at[pl.ds(i * C, C), :],
                            out_hbm.at[pl.ds(c * N + i * C, C), :])


@functools.cache
def _pool_kernel():
    return pl.kernel(
        _pool_body,
        out_type=[jax.ShapeDtypeStruct((NW, 2, L * H), jnp.float32),
                  jax.ShapeDtypeStruct((NW, 2, L * H), jnp.float32)],
        mesh=_mesh(),
        scratch_types=[
            pltpu.VMEM((N,), jnp.int32),
            pltpu.VMEM((PCP, H), jnp.float32),
            pltpu.VMEM((PCP, H), jnp.float32),
            pltpu.VMEM((PCP, H), jnp.float32),
            pltpu.VMEM((L * H,), jnp.float32),
            pltpu.VMEM((L * H,), jnp.float32),
            pltpu.VMEM((2, L * H), jnp.float32),
            pltpu.VMEM((2, L * H), jnp.float32),
        ],
        compiler_params=_SC_PARAMS,
    )


def _pool_body(h1_hbm, h2_hbm, h3_hbm, b_hbm, mean_hbm, max_hbm,
               bv, r1, r2, r3, sacc, macc, mrow, xrow):
    c = lax.axis_index("c")
    s = lax.axis_index("s")
    w = c * NS + s
    pltpu.sync_copy(b_hbm, bv)
    hs = (h1_hbm, h2_hbm, h3_hbm)
    rs = (r1, r2, r3)

    for gl in range(2):
        g = 2 * w + gl

        def cnt(i, carry):
            sg, eg = carry
            v = bv[pl.ds(VL * i, VL)]
            sg = sg + jnp.sum((v < g).astype(jnp.int32))
            eg = eg + jnp.sum((v <= g).astype(jnp.int32))
            return sg, eg

        sg, eg = lax.fori_loop(0, N // VL, cnt,
                               (jnp.int32(0), jnp.int32(0)))

        for j in range(L * H // VL):
            sacc[pl.ds(VL * j, VL)] = jnp.zeros((VL,), jnp.float32)
            macc[pl.ds(VL * j, VL)] = jnp.full((VL,), -jnp.inf, jnp.float32)

        k0 = sg // PCP
        k1 = (eg + PCP - 1) // PCP

        def chunk(k, carry):
            rbase = k * PCP
            for t in range(L):
                pltpu.sync_copy(hs[t].at[pl.ds(rbase, PCP), :], rs[t])

            def row(i, inner_carry):
                ridx = jnp.full((VL,), rbase + i, jnp.int32)
                valid = (ridx >= sg) & (ridx < eg)
                for t in range(L):
                    for j in range(H // VL):
                        v = rs[t][i, pl.ds(VL * j, VL)]
                        asl = pl.ds(t * H + VL * j, VL)
                        sacc[asl] = sacc[asl] + jnp.where(valid, v, 0.0)
                        macc[asl] = jnp.maximum(
                            macc[asl], jnp.where(valid, v, -jnp.inf))
                return inner_carry

            lax.fori_loop(0, PCP, row, 0)
            return carry

        lax.fori_loop(k0, k1, chunk, 0)
        denom = jnp.maximum((eg - sg).astype(jnp.float32), 1.0)
        for j in range(L * H // VL):
            sl = pl.ds(VL * j, VL)
            mrow[gl, sl] = sacc[sl] / denom
            xrow[gl, sl] = macc[sl]

    pltpu.sync_copy(mrow, mean_hbm.at[w])
    pltpu.sync_copy(xrow, max_hbm.at[w])


def kernel(x, edge_index, edge_attr, batch, W_in, b_in, We, be, W1, b1,
           gm, bm, W2, b2, go, bo, eps, Wh1, bh1, gh, bh, Wh2, bh2):
    src = edge_index[0].astype(jnp.int32)
    dst = edge_index[1].astype(jnp.int32)
    batch32 = batch.astype(jnp.int32)

    s1 = gm / jnp.sqrt(1.0 + BN_EPS)
    w1f = W1 * s1[:, None, :]
    b1f = b1 * s1 + bm
    s2 = go / jnp.sqrt(1.0 + BN_EPS)
    w2f = W2 * s2[:, None, :]
    b2f = b2 * s2 + bo
    sh = gh / jnp.sqrt(1.0 + BN_EPS)
    wh1f = Wh1 * sh[None, :]
    bh1f = bh1 * sh + bh

    h = _linear(x, W_in, b_in)
    ea4 = edge_attr.astype(jnp.float32).reshape(E // C, C // 4, 4 * ED)

    outs = []
    for l in range(L):
        parts = _edge_pass_kernel()(src, dst, ea4, We[l], be[l], h)
        scale_row = jnp.broadcast_to(
            (1.0 + eps[l]).astype(jnp.float32)[None, None], (1, H))
        h = _mlp(h, parts.reshape(NC, N, H), scale_row,
                 w1f[l], b1f[l], w2f[l], b2f[l])
        outs.append(h)

    means, maxes = _pool_kernel()(outs[0], outs[1], outs[2], batch32)
    o = _head(means.reshape(G, L * H), maxes.reshape(G, L * H),
              wh1f[:L * H], wh1f[L * H:], bh1f, Wh2, bh2)
    return o[:, 0]


# speedup vs baseline: 3.0358x; 1.2273x over previous
import functools

import jax
import jax.numpy as jnp
from jax import lax
from jax.experimental import pallas as pl
from jax.experimental.pallas import tpu as pltpu
from jax.experimental.pallas import tpu_sc as plsc

N = 10000
E = 320000
D = 128
H = 128
ED = 4
L = 3
G = 64
BN_EPS = 1e-5

NC = 2
NS = 16
NW = NC * NS
EPW = E // NW
C = 80
NCH = EPW // C
RPS = N // NS
ZR = 125
VL = 16
PCP = 16

@functools.cache
def _mesh():
    return plsc.VectorSubcoreMesh(
        core_axis_name="c", subcore_axis_name="s",
        num_cores=NC, num_subcores=NS)


_SC_PARAMS = pltpu.CompilerParams(needs_layout_passes=False)


def _lin_body(x_ref, w_ref, b_ref, o_ref):
    o_ref[...] = (
        jnp.dot(x_ref[...], w_ref[...], preferred_element_type=jnp.float32)
        + b_ref[...])


def _linear(x, w, b, bm=1000):
    n, d = x.shape
    h = w.shape[1]
    return pl.pallas_call(
        _lin_body,
        grid=(n // bm,),
        in_specs=[
            pl.BlockSpec((bm, d), lambda i: (i, 0)),
            pl.BlockSpec((d, h), lambda i: (0, 0)),
            pl.BlockSpec((1, h), lambda i: (0, 0)),
        ],
        out_specs=pl.BlockSpec((bm, h), lambda i: (i, 0)),
        out_shape=jax.ShapeDtypeStruct((n, h), jnp.float32),
    )(x, w, b.reshape(1, h))


def _mlp_body(h_ref, p_ref, sc_ref, w1_ref, b1_ref, w2_ref, b2_ref, o_ref):
    z = sc_ref[...] * h_ref[...] + p_ref[0] + p_ref[1]
    z = jnp.dot(z, w1_ref[...], preferred_element_type=jnp.float32) + b1_ref[...]
    z = jnp.maximum(z, 0.0)
    z = jnp.dot(z, w2_ref[...], preferred_element_type=jnp.float32) + b2_ref[...]
    o_ref[...] = jnp.maximum(z, 0.0)


def _mlp(h, parts, scale_row, w1, b1, w2, b2, bm=1000):
    return pl.pallas_call(
        _mlp_body,
        grid=(N // bm,),
        in_specs=[
            pl.BlockSpec((bm, H), lambda i: (i, 0)),
            pl.BlockSpec((NC, bm, H), lambda i: (0, i, 0)),
            pl.BlockSpec((1, H), lambda i: (0, 0)),
            pl.BlockSpec((H, H), lambda i: (0, 0)),
            pl.BlockSpec((1, H), lambda i: (0, 0)),
            pl.BlockSpec((H, H), lambda i: (0, 0)),
            pl.BlockSpec((1, H), lambda i: (0, 0)),
        ],
        out_specs=pl.BlockSpec((bm, H), lambda i: (i, 0)),
        out_shape=jax.ShapeDtypeStruct((N, H), jnp.float32),
    )(h, parts, scale_row, w1, b1.reshape(1, H), w2, b2.reshape(1, H))


def _head_body(mn_ref, mx_ref, a1_ref, a2_ref, b1_ref, w2_ref, b2_ref, o_ref):
    t = (jnp.dot(mn_ref[...], a1_ref[...], preferred_element_type=jnp.float32)
         + jnp.dot(mx_ref[...], a2_ref[...], preferred_element_type=jnp.float32)
         + b1_ref[...])
    t = jnp.maximum(t, 0.0)
    o_ref[...] = (
        jnp.dot(t, w2_ref[...], preferred_element_type=jnp.float32) + b2_ref[...])


def _head(means, maxes, a1, a2, b1, w2, b2):
    return pl.pallas_call(
        _head_body,
        out_shape=jax.ShapeDtypeStruct((G, 1), jnp.float32),
    )(means, maxes, a1, a2, b1.reshape(1, H), w2, b2.reshape(1, 1))


@functools.cache
def _edge_pass_kernel():
    return pl.kernel(
        _edge_pass_body,
        out_type=jax.ShapeDtypeStruct((NC * N, H), jnp.float32),
        mesh=_mesh(),
        scratch_types=[
            pltpu.VMEM((C,), jnp.int32),
            pltpu.VMEM((C,), jnp.int32),
            pltpu.VMEM((C,), jnp.int32),
            pltpu.VMEM((C,), jnp.int32),
            pltpu.VMEM((C, H), jnp.float32),
            pltpu.VMEM((C, H), jnp.float32),
            pltpu.VMEM((C // 4, VL), jnp.float32),
            pltpu.VMEM((C // 4, VL), jnp.float32),
            pltpu.VMEM((ED, H), jnp.float32),
            pltpu.VMEM((H,), jnp.float32),
            pltpu.VMEM((C, H), jnp.float32),
            pltpu.VMEM_SHARED((N, H), jnp.float32),
            pltpu.SemaphoreType.DMA,
            pltpu.SemaphoreType.DMA,
        ],
        compiler_params=_SC_PARAMS,
    )


def _edge_pass_body(src_hbm, dst_hbm, ea_hbm, we_hbm, be_hbm, h_hbm, out_hbm,
                    isa, isb, ida, idb, rowsa, rowsb, eava, eavb,
                    wev, bev, zbuf, aggr, gsa, gsb):
    c = lax.axis_index("c")
    s = lax.axis_index("s")
    w = c * NS + s
    pltpu.sync_copy(we_hbm, wev)
    pltpu.sync_copy(be_hbm, bev)

    def zrow(i, carry):
        for j in range(H // VL):
            zbuf[i, pl.ds(VL * j, VL)] = jnp.zeros((VL,), jnp.float32)
        return carry

    lax.fori_loop(0, C, zrow, 0)
    for t in range((N // C + NS - 1) // NS):
        i = s + NS * t
        @pl.when(i < N // C)
        def _():
            pltpu.sync_copy(zbuf, aggr.at[pl.ds(i * C, C), :])
    plsc.subcore_barrier()

    base = pl.multiple_of(w * EPW, 8)
    wk = [[wev[k, pl.ds(VL * j, VL)] for j in range(H // VL)]
          for k in range(ED)]
    bb = [bev[pl.ds(VL * j, VL)] for j in range(H // VL)]

    def fetch(k, is_, id_, ea_):
        eb = pl.multiple_of(base + k * C, 8)
        pltpu.sync_copy(src_hbm.at[pl.ds(eb, C)], is_)
        pltpu.sync_copy(dst_hbm.at[pl.ds(eb, C)], id_)
        pltpu.sync_copy(ea_hbm.at[w * NCH + k], ea_)

    def start_gather(is_, rows_, sem_):
        pltpu.async_copy(h_hbm.at[is_], rows_, sem_)

    def wait_gather(is_, rows_, sem_):
        pltpu.make_async_copy(h_hbm.at[is_], rows_, sem_).wait()

    def compute(rows_, ea_):
        def quad(q, inner_carry):
            av = ea_[q, :]
            for je in range(4):
                i = 4 * q + je
                a = [av[4 * je + kk] for kk in range(ED)]
                for j in range(H // VL):
                    sl = pl.ds(VL * j, VL)
                    e = bb[j]
                    for kk in range(ED):
                        e = e + a[kk] * wk[kk][j]
                    rows_[i, sl] = jnp.maximum(rows_[i, sl] + e, 0.0)
            return inner_carry

        lax.fori_loop(0, C // 4, quad, 0)

    def scatter(rows_, id_):
        pltpu.sync_copy(rows_, aggr.at[id_], add=True)

    fetch(0, isa, ida, eava)
    start_gather(isa, rowsa, gsa)

    def pair(t, carry):
        k0 = 2 * t
        fetch(k0 + 1, isb, idb, eavb)
        start_gather(isb, rowsb, gsb)
        wait_gather(isa, rowsa, gsa)
        compute(rowsa, eava)
        scatter(rowsa, ida)
        fetch(k0 + 2, isa, ida, eava)
        start_gather(isa, rowsa, gsa)
        wait_gather(isb, rowsb, gsb)
        compute(rowsb, eavb)
        scatter(rowsb, idb)
        return carry

    lax.fori_loop(0, (NCH - 1) // 2, pair, 0)
    wait_gather(isa, rowsa, gsa)
    compute(rowsa, eava)
    scatter(rowsa, ida)
    plsc.subcore_barrier()
    for t in range((N // C + NS - 1) // NS):
        i = s + NS * t
        @pl.when(i < N // C)
        def _():
            pltpu.sync_copy(aggr.at[pl.ds(i * C, C), :],
                            out_hbm.at[pl.ds(c * N + i * C, C), :])


@functools.cache
def _pool_kernel():
    return pl.kernel(
        _pool_body,
        out_type=[jax.ShapeDtypeStruct((NW, 2, L * H), jnp.float32),
                  jax.ShapeDtypeStruct((NW, 2, L * H), jnp.float32)],
        mesh=_mesh(),
        scratch_types=[
            pltpu.VMEM((N,), jnp.int32),
            pltpu.VMEM((PCP, H), jnp.float32),
            pltpu.VMEM((PCP, H), jnp.float32),
            pltpu.VMEM((PCP, H), jnp.float32),
            pltpu.VMEM((L * H,), jnp.float32),
            pltpu.VMEM((L * H,), jnp.float32),
            pltpu.VMEM((2, L * H), jnp.float32),
            pltpu.VMEM((2, L * H), jnp.float32),
        ],
        compiler_params=_SC_PARAMS,
    )


def _pool_body(h1_hbm, h2_hbm, h3_hbm, b_hbm, mean_hbm, max_hbm,
               bv, r1, r2, r3, sacc, macc, mrow, xrow):
    c = lax.axis_index("c")
    s = lax.axis_index("s")
    w = c * NS + s
    pltpu.sync_copy(b_hbm, bv)
    hs = (h1_hbm, h2_hbm, h3_hbm)
    rs = (r1, r2, r3)

    for gl in range(2):
        g = 2 * w + gl

        def cnt(i, carry):
            sg, eg = carry
            v = bv[pl.ds(VL * i, VL)]
            sg = sg + jnp.sum((v < g).astype(jnp.int32))
            eg = eg + jnp.sum((v <= g).astype(jnp.int32))
            return sg, eg

        sg, eg = lax.fori_loop(0, N // VL, cnt,
                               (jnp.int32(0), jnp.int32(0)))

        for j in range(L * H // VL):
            sacc[pl.ds(VL * j, VL)] = jnp.zeros((VL,), jnp.float32)
            macc[pl.ds(VL * j, VL)] = jnp.full((VL,), -jnp.inf, jnp.float32)

        k0 = sg // PCP
        k1 = (eg + PCP - 1) // PCP

        def chunk(k, carry):
            rbase = k * PCP
            for t in range(L):
                pltpu.sync_copy(hs[t].at[pl.ds(rbase, PCP), :], rs[t])

            def row(i, inner_carry):
                ridx = jnp.full((VL,), rbase + i, jnp.int32)
                valid = (ridx >= sg) & (ridx < eg)
                for t in range(L):
                    for j in range(H // VL):
                        v = rs[t][i, pl.ds(VL * j, VL)]
                        asl = pl.ds(t * H + VL * j, VL)
                        sacc[asl] = sacc[asl] + jnp.where(valid, v, 0.0)
                        macc[asl] = jnp.maximum(
                            macc[asl], jnp.where(valid, v, -jnp.inf))
                return inner_carry

            lax.fori_loop(0, PCP, row, 0)
            return carry

        lax.fori_loop(k0, k1, chunk, 0)
        denom = jnp.maximum((eg - sg).astype(jnp.float32), 1.0)
        for j in range(L * H // VL):
            sl = pl.ds(VL * j, VL)
            mrow[gl, sl] = sacc[sl] / denom
            xrow[gl, sl] = macc[sl]

    pltpu.sync_copy(mrow, mean_hbm.at[w])
    pltpu.sync_copy(xrow, max_hbm.at[w])


def kernel(x, edge_index, edge_attr, batch, W_in, b_in, We, be, W1, b1,
           gm, bm, W2, b2, go, bo, eps, Wh1, bh1, gh, bh, Wh2, bh2):
    src = edge_index[0].astype(jnp.int32)
    dst = edge_index[1].astype(jnp.int32)
    batch32 = batch.astype(jnp.int32)

    s1 = gm / jnp.sqrt(1.0 + BN_EPS)
    w1f = W1 * s1[:, None, :]
    b1f = b1 * s1 + bm
    s2 = go / jnp.sqrt(1.0 + BN_EPS)
    w2f = W2 * s2[:, None, :]
    b2f = b2 * s2 + bo
    sh = gh / jnp.sqrt(1.0 + BN_EPS)
    wh1f = Wh1 * sh[None, :]
    bh1f = bh1 * sh + bh

    h = _linear(x, W_in, b_in)
    ea4 = edge_attr.astype(jnp.float32).reshape(E // C, C // 4, 4 * ED)

    outs = []
    for l in range(L):
        parts = _edge_pass_kernel()(src, dst, ea4, We[l], be[l], h)
        scale_row = jnp.broadcast_to(
            (1.0 + eps[l]).astype(jnp.float32)[None, None], (1, H))
        h = _mlp(h, parts.reshape(NC, N, H), scale_row,
                 w1f[l], b1f[l], w2f[l], b2f[l])
        outs.append(h)

    means, maxes = _pool_kernel()(outs[0], outs[1], outs[2], batch32)
    o = _head(means.reshape(G, L * H), maxes.reshape(G, L * H),
              wh1f[:L * H], wh1f[L * H:], bh1f, Wh2, bh2)
    return o[:, 0]


# speedup vs baseline: 4.4733x; 1.4735x over previous
import functools

import jax
import jax.numpy as jnp
from jax import lax
from jax.experimental import pallas as pl
from jax.experimental.pallas import tpu as pltpu
from jax.experimental.pallas import tpu_sc as plsc

N = 10000
E = 320000
D = 128
H = 128
ED = 4
L = 3
G = 64
BN_EPS = 1e-5

NC = 2
NS = 16
NW = NC * NS
EPW = E // NW
C = 80
NCH = EPW // C
RPS = N // NS
ZR = 125
VL = 16
PCP = 16

@functools.cache
def _mesh():
    return plsc.VectorSubcoreMesh(
        core_axis_name="c", subcore_axis_name="s",
        num_cores=NC, num_subcores=NS)


_SC_PARAMS = pltpu.CompilerParams(
    needs_layout_passes=False, use_tc_tiling_on_sc=False)


def _lin_body(x_ref, w_ref, b_ref, o_ref):
    o_ref[...] = (
        jnp.dot(x_ref[...], w_ref[...], preferred_element_type=jnp.float32)
        + b_ref[...])


def _linear(x, w, b, bm=1000):
    n, d = x.shape
    h = w.shape[1]
    return pl.pallas_call(
        _lin_body,
        grid=(n // bm,),
        in_specs=[
            pl.BlockSpec((bm, d), lambda i: (i, 0)),
            pl.BlockSpec((d, h), lambda i: (0, 0)),
            pl.BlockSpec((1, h), lambda i: (0, 0)),
        ],
        out_specs=pl.BlockSpec((bm, h), lambda i: (i, 0)),
        out_shape=jax.ShapeDtypeStruct((n, h), jnp.float32),
    )(x, w, b.reshape(1, h))


def _mlp_body(h_ref, p_ref, sc_ref, w1_ref, b1_ref, w2_ref, b2_ref, o_ref):
    z = sc_ref[...] * h_ref[...] + p_ref[0] + p_ref[1]
    z = jnp.dot(z, w1_ref[...], preferred_element_type=jnp.float32) + b1_ref[...]
    z = jnp.maximum(z, 0.0)
    z = jnp.dot(z, w2_ref[...], preferred_element_type=jnp.float32) + b2_ref[...]
    o_ref[...] = jnp.maximum(z, 0.0)


def _mlp(h, parts, scale_row, w1, b1, w2, b2, bm=1000):
    return pl.pallas_call(
        _mlp_body,
        grid=(N // bm,),
        in_specs=[
            pl.BlockSpec((bm, H), lambda i: (i, 0)),
            pl.BlockSpec((NC, bm, H), lambda i: (0, i, 0)),
            pl.BlockSpec((1, H), lambda i: (0, 0)),
            pl.BlockSpec((H, H), lambda i: (0, 0)),
            pl.BlockSpec((1, H), lambda i: (0, 0)),
            pl.BlockSpec((H, H), lambda i: (0, 0)),
            pl.BlockSpec((1, H), lambda i: (0, 0)),
        ],
        out_specs=pl.BlockSpec((bm, H), lambda i: (i, 0)),
        out_shape=jax.ShapeDtypeStruct((N, H), jnp.float32),
    )(h, parts, scale_row, w1, b1.reshape(1, H), w2, b2.reshape(1, H))


def _head_body(mn_ref, mx_ref, a1_ref, a2_ref, b1_ref, w2_ref, b2_ref, o_ref):
    t = (jnp.dot(mn_ref[...], a1_ref[...], preferred_element_type=jnp.float32)
         + jnp.dot(mx_ref[...], a2_ref[...], preferred_element_type=jnp.float32)
         + b1_ref[...])
    t = jnp.maximum(t, 0.0)
    o_ref[...] = (
        jnp.dot(t, w2_ref[...], preferred_element_type=jnp.float32) + b2_ref[...])


def _head(means, maxes, a1, a2, b1, w2, b2):
    return pl.pallas_call(
        _head_body,
        out_shape=jax.ShapeDtypeStruct((G, 1), jnp.float32),
    )(means, maxes, a1, a2, b1.reshape(1, H), w2, b2.reshape(1, 1))


@functools.cache
def _edge_pass_kernel():
    return pl.kernel(
        _edge_pass_body,
        out_type=jax.ShapeDtypeStruct((NC * N, H), jnp.float32),
        mesh=_mesh(),
        scratch_types=[
            pltpu.VMEM((NCH, C), jnp.int32),
            pltpu.VMEM((NCH, C), jnp.int32),
            pltpu.VMEM((C // 4, VL), jnp.float32),
            pltpu.VMEM((C // 4, VL), jnp.float32),
            pltpu.VMEM((C, H), jnp.float32),
            pltpu.VMEM((C, H), jnp.float32),
            pltpu.VMEM((ED, H), jnp.float32),
            pltpu.VMEM((H,), jnp.float32),
            pltpu.VMEM_SHARED((N, H), jnp.float32),
            pltpu.SemaphoreType.DMA,
            pltpu.SemaphoreType.DMA,
            pltpu.SemaphoreType.DMA,
            pltpu.SemaphoreType.DMA,
        ],
        compiler_params=_SC_PARAMS,
    )


def _edge_pass_body(src_hbm, dst_hbm, ea_hbm, we_hbm, be_hbm, h_hbm, out_hbm,
                    isv, idv, eava, eavb, rowsa, rowsb,
                    wev, bev, aggr, gsa, gsb, esa, esb):
    c = lax.axis_index("c")
    s = lax.axis_index("s")
    w = c * NS + s
    pltpu.sync_copy(we_hbm, wev)
    pltpu.sync_copy(be_hbm, bev)
    pltpu.sync_copy(src_hbm.at[w], isv)
    pltpu.sync_copy(dst_hbm.at[w], idv)

    def zrow(i, carry):
        for j in range(H // VL):
            rowsa[i, pl.ds(VL * j, VL)] = jnp.zeros((VL,), jnp.float32)
        return carry

    lax.fori_loop(0, C, zrow, 0)
    for t in range((N // C + NS - 1) // NS):
        i = s + NS * t
        @pl.when(i < N // C)
        def _():
            pltpu.sync_copy(rowsa, aggr.at[pl.ds(i * C, C), :])
    plsc.subcore_barrier()

    wk = [[wev[k, pl.ds(VL * j, VL)] for j in range(H // VL)]
          for k in range(ED)]
    bb = [bev[pl.ds(VL * j, VL)] for j in range(H // VL)]

    def start_gather(k, rows_, sem_):
        pltpu.async_copy(h_hbm.at[isv.at[k]], rows_, sem_)

    def wait_gather(k, rows_, sem_):
        pltpu.make_async_copy(h_hbm.at[isv.at[k]], rows_, sem_).wait()

    def start_ea(k, ea_, sem_):
        pltpu.async_copy(ea_hbm.at[w * NCH + k], ea_, sem_)

    def wait_ea(k, ea_, sem_):
        pltpu.make_async_copy(ea_hbm.at[w * NCH + k], ea_, sem_).wait()

    def compute(rows_, eav):
        def quad(q, inner_carry):
            av = eav[q, :]
            for je in range(4):
                i = 4 * q + je
                a = [av[4 * je + kk] for kk in range(ED)]
                for j in range(H // VL):
                    sl = pl.ds(VL * j, VL)
                    e = bb[j]
                    for kk in range(ED):
                        e = e + a[kk] * wk[kk][j]
                    rows_[i, sl] = jnp.maximum(rows_[i, sl] + e, 0.0)
            return inner_carry

        lax.fori_loop(0, C // 4, quad, 0)

    def scatter(k, rows_):
        pltpu.sync_copy(rows_, aggr.at[idv.at[k]], add=True)

    start_ea(0, eava, esa)
    start_gather(0, rowsa, gsa)

    def pair(t, carry):
        k0 = 2 * t
        start_ea(k0 + 1, eavb, esb)
        start_gather(k0 + 1, rowsb, gsb)
        wait_gather(k0, rowsa, gsa)
        wait_ea(k0, eava, esa)
        compute(rowsa, eava)
        scatter(k0, rowsa)
        start_ea(k0 + 2, eava, esa)
        start_gather(k0 + 2, rowsa, gsa)
        wait_gather(k0 + 1, rowsb, gsb)
        wait_ea(k0 + 1, eavb, esb)
        compute(rowsb, eavb)
        scatter(k0 + 1, rowsb)
        return carry

    lax.fori_loop(0, (NCH - 1) // 2, pair, 0)
    wait_gather(NCH - 1, rowsa, gsa)
    wait_ea(NCH - 1, eava, esa)
    compute(rowsa, eava)
    scatter(NCH - 1, rowsa)
    plsc.subcore_barrier()
    for t in range((N // C + NS - 1) // NS):
        i = s + NS * t
        @pl.when(i < N // C)
        def _():
            pltpu.sync_copy(aggr.at[pl.ds(i * C, C), :],
                            out_hbm.at[pl.ds(c * N + i * C, C), :])


@functools.cache
def _pool_kernel():
    return pl.kernel(
        _pool_body,
        out_type=[jax.ShapeDtypeStruct((NW, 2, L * H), jnp.float32),
                  jax.ShapeDtypeStruct((NW, 2, L * H), jnp.float32)],
        mesh=_mesh(),
        scratch_types=[
            pltpu.VMEM((N,), jnp.int32),
            pltpu.VMEM((PCP, H), jnp.float32),
            pltpu.VMEM((PCP, H), jnp.float32),
            pltpu.VMEM((PCP, H), jnp.float32),
            pltpu.VMEM((L * H,), jnp.float32),
            pltpu.VMEM((L * H,), jnp.float32),
            pltpu.VMEM((2, L * H), jnp.float32),
            pltpu.VMEM((2, L * H), jnp.float32),
        ],
        compiler_params=_SC_PARAMS,
    )


def _pool_body(h1_hbm, h2_hbm, h3_hbm, b_hbm, mean_hbm, max_hbm,
               bv, r1, r2, r3, sacc, macc, mrow, xrow):
    c = lax.axis_index("c")
    s = lax.axis_index("s")
    w = c * NS + s
    pltpu.sync_copy(b_hbm, bv)
    hs = (h1_hbm, h2_hbm, h3_hbm)
    rs = (r1, r2, r3)

    for gl in range(2):
        g = 2 * w + gl

        def cnt(i, carry):
            sg, eg = carry
            v = bv[pl.ds(VL * i, VL)]
            sg = sg + jnp.sum((v < g).astype(jnp.int32))
            eg = eg + jnp.sum((v <= g).astype(jnp.int32))
            return sg, eg

        sg, eg = lax.fori_loop(0, N // VL, cnt,
                               (jnp.int32(0), jnp.int32(0)))

        for j in range(L * H // VL):
            sacc[pl.ds(VL * j, VL)] = jnp.zeros((VL,), jnp.float32)
            macc[pl.ds(VL * j, VL)] = jnp.full((VL,), -jnp.inf, jnp.float32)

        k0 = sg // PCP
        k1 = (eg + PCP - 1) // PCP

        def chunk(k, carry):
            rbase = k * PCP
            for t in range(L):
                pltpu.sync_copy(hs[t].at[pl.ds(rbase, PCP), :], rs[t])

            def row(i, inner_carry):
                ridx = jnp.full((VL,), rbase + i, jnp.int32)
                valid = (ridx >= sg) & (ridx < eg)
                for t in range(L):
                    for j in range(H // VL):
                        v = rs[t][i, pl.ds(VL * j, VL)]
                        asl = pl.ds(t * H + VL * j, VL)
                        sacc[asl] = sacc[asl] + jnp.where(valid, v, 0.0)
                        macc[asl] = jnp.maximum(
                            macc[asl], jnp.where(valid, v, -jnp.inf))
                return inner_carry

            lax.fori_loop(0, PCP, row, 0)
            return carry

        lax.fori_loop(k0, k1, chunk, 0)
        denom = jnp.maximum((eg - sg).astype(jnp.float32), 1.0)
        for j in range(L * H // VL):
            sl = pl.ds(VL * j, VL)
            mrow[gl, sl] = sacc[sl] / denom
            xrow[gl, sl] = macc[sl]

    pltpu.sync_copy(mrow, mean_hbm.at[w])
    pltpu.sync_copy(xrow, max_hbm.at[w])


def kernel(x, edge_index, edge_attr, batch, W_in, b_in, We, be, W1, b1,
           gm, bm, W2, b2, go, bo, eps, Wh1, bh1, gh, bh, Wh2, bh2):
    src = edge_index[0].astype(jnp.int32)
    dst = edge_index[1].astype(jnp.int32)
    batch32 = batch.astype(jnp.int32)

    s1 = gm / jnp.sqrt(1.0 + BN_EPS)
    w1f = W1 * s1[:, None, :]
    b1f = b1 * s1 + bm
    s2 = go / jnp.sqrt(1.0 + BN_EPS)
    w2f = W2 * s2[:, None, :]
    b2f = b2 * s2 + bo
    sh = gh / jnp.sqrt(1.0 + BN_EPS)
    wh1f = Wh1 * sh[None, :]
    bh1f = bh1 * sh + bh

    h = _linear(x, W_in, b_in)
    src3 = src.reshape(NW, NCH, C)
    dst3 = dst.reshape(NW, NCH, C)
    ea4 = edge_attr.astype(jnp.float32).reshape(NW * NCH, C // 4, 4 * ED)

    outs = []
    for l in range(L):
        parts = _edge_pass_kernel()(src3, dst3, ea4, We[l], be[l], h)
        scale_row = jnp.broadcast_to(
            (1.0 + eps[l]).astype(jnp.float32)[None, None], (1, H))
        h = _mlp(h, parts.reshape(NC, N, H), scale_row,
                 w1f[l], b1f[l], w2f[l], b2f[l])
        outs.append(h)

    means, maxes = _pool_kernel()(outs[0], outs[1], outs[2], batch32)
    o = _head(means.reshape(G, L * H), maxes.reshape(G, L * H),
              wh1f[:L * H], wh1f[L * H:], bh1f, Wh2, bh2)
    return o[:, 0]


# speedup vs baseline: 5.1646x; 1.1546x over previous
import functools

import jax
import jax.numpy as jnp
from jax import lax
from jax.experimental import pallas as pl
from jax.experimental.pallas import tpu as pltpu
from jax.experimental.pallas import tpu_sc as plsc

N = 10000
E = 320000
D = 128
H = 128
ED = 4
L = 3
G = 64
BN_EPS = 1e-5

NC = 2
NS = 16
NW = NC * NS
EPW = E // NW
C = 80
NCH = EPW // C
RPS = N // NS
ZR = 125
VL = 16
PCP = 16

@functools.cache
def _mesh():
    return plsc.VectorSubcoreMesh(
        core_axis_name="c", subcore_axis_name="s",
        num_cores=NC, num_subcores=NS)


_SC_PARAMS = pltpu.CompilerParams(
    needs_layout_passes=False, use_tc_tiling_on_sc=False)


def _lin_body(x_ref, w_ref, b_ref, o_ref):
    o_ref[...] = (
        jnp.dot(x_ref[...], w_ref[...], preferred_element_type=jnp.float32)
        + b_ref[...])


def _linear(x, w, b, bm=1000):
    n, d = x.shape
    h = w.shape[1]
    return pl.pallas_call(
        _lin_body,
        grid=(n // bm,),
        in_specs=[
            pl.BlockSpec((bm, d), lambda i: (i, 0)),
            pl.BlockSpec((d, h), lambda i: (0, 0)),
            pl.BlockSpec((1, h), lambda i: (0, 0)),
        ],
        out_specs=pl.BlockSpec((bm, h), lambda i: (i, 0)),
        out_shape=jax.ShapeDtypeStruct((n, h), jnp.float32),
    )(x, w, b.reshape(1, h))


def _mlp_body(h_ref, p_ref, sc_ref, w1_ref, b1_ref, w2_ref, b2_ref, o_ref):
    z = sc_ref[...] * h_ref[...] + p_ref[0] + p_ref[1]
    z = jnp.dot(z, w1_ref[...], preferred_element_type=jnp.float32) + b1_ref[...]
    z = jnp.maximum(z, 0.0)
    z = jnp.dot(z, w2_ref[...], preferred_element_type=jnp.float32) + b2_ref[...]
    o_ref[...] = jnp.maximum(z, 0.0)


def _mlp(h, parts, scale_row, w1, b1, w2, b2, bm=1000):
    return pl.pallas_call(
        _mlp_body,
        grid=(N // bm,),
        in_specs=[
            pl.BlockSpec((bm, H), lambda i: (i, 0)),
            pl.BlockSpec((NC, bm, H), lambda i: (0, i, 0)),
            pl.BlockSpec((1, H), lambda i: (0, 0)),
            pl.BlockSpec((H, H), lambda i: (0, 0)),
            pl.BlockSpec((1, H), lambda i: (0, 0)),
            pl.BlockSpec((H, H), lambda i: (0, 0)),
            pl.BlockSpec((1, H), lambda i: (0, 0)),
        ],
        out_specs=pl.BlockSpec((bm, H), lambda i: (i, 0)),
        out_shape=jax.ShapeDtypeStruct((N, H), jnp.float32),
    )(h, parts, scale_row, w1, b1.reshape(1, H), w2, b2.reshape(1, H))


def _head_body(mn_ref, mx_ref, a1_ref, a2_ref, b1_ref, w2_ref, b2_ref, o_ref):
    t = (jnp.dot(mn_ref[...], a1_ref[...], preferred_element_type=jnp.float32)
         + jnp.dot(mx_ref[...], a2_ref[...], preferred_element_type=jnp.float32)
         + b1_ref[...])
    t = jnp.maximum(t, 0.0)
    o_ref[...] = (
        jnp.dot(t, w2_ref[...], preferred_element_type=jnp.float32) + b2_ref[...])


def _head(means, maxes, a1, a2, b1, w2, b2):
    return pl.pallas_call(
        _head_body,
        out_shape=jax.ShapeDtypeStruct((G, 1), jnp.float32),
    )(means, maxes, a1, a2, b1.reshape(1, H), w2, b2.reshape(1, 1))


@functools.cache
def _edge_pass_kernel():
    return pl.kernel(
        _edge_pass_body,
        out_type=jax.ShapeDtypeStruct((NC * N, H), jnp.float32),
        mesh=_mesh(),
        scratch_types=[
            pltpu.VMEM((NCH, C), jnp.int32),
            [pltpu.VMEM((C,), jnp.int32) for _ in range(3)],
            [pltpu.VMEM((C // 4, VL), jnp.float32) for _ in range(3)],
            [pltpu.VMEM((C, H), jnp.float32) for _ in range(3)],
            pltpu.VMEM((ED, H), jnp.float32),
            pltpu.VMEM((H,), jnp.float32),
            pltpu.VMEM_SHARED((N, H), jnp.float32),
            [pltpu.SemaphoreType.DMA for _ in range(3)],
            [pltpu.SemaphoreType.DMA for _ in range(3)],
            [pltpu.SemaphoreType.DMA for _ in range(3)],
        ],
        compiler_params=_SC_PARAMS,
    )


def _edge_pass_body(src_hbm, dst_hbm, ea_hbm, we_hbm, be_hbm, h_hbm, out_hbm,
                    idv, ISV, EAV, ROWS, wev, bev, aggr, GS, SS, FS):
    c = lax.axis_index("c")
    s = lax.axis_index("s")
    w = c * NS + s
    pltpu.sync_copy(we_hbm, wev)
    pltpu.sync_copy(be_hbm, bev)
    pltpu.sync_copy(dst_hbm.at[w], idv)

    def zrow(i, carry):
        for j in range(H // VL):
            ROWS[0][i, pl.ds(VL * j, VL)] = jnp.zeros((VL,), jnp.float32)
        return carry

    lax.fori_loop(0, C, zrow, 0)
    for t in range((N // C + NS - 1) // NS):
        i = s + NS * t
        @pl.when(i < N // C)
        def _():
            pltpu.sync_copy(ROWS[0], aggr.at[pl.ds(i * C, C), :])
    plsc.subcore_barrier()

    wk = [[wev[k, pl.ds(VL * j, VL)] for j in range(H // VL)]
          for k in range(ED)]
    bb = [bev[pl.ds(VL * j, VL)] for j in range(H // VL)]

    def start_fetch(k, j):
        pltpu.async_copy(src_hbm.at[w * NCH + k], ISV[j], FS[j])
        pltpu.async_copy(ea_hbm.at[w * NCH + k], EAV[j], FS[j])

    def wait_fetch(j):
        pltpu.make_async_copy(src_hbm.at[0], ISV[j], FS[j]).wait()
        pltpu.make_async_copy(ea_hbm.at[0], EAV[j], FS[j]).wait()

    def start_gather(j):
        pltpu.async_copy(h_hbm.at[ISV[j]], ROWS[j], GS[j])

    def wait_gather(j):
        pltpu.make_async_copy(h_hbm.at[ISV[j]], ROWS[j], GS[j]).wait()

    def start_scatter(k, j):
        pltpu.async_copy(ROWS[j], aggr.at[idv.at[k]], SS[j], add=True)

    def wait_scatter(k, j):
        pltpu.make_async_copy(ROWS[j], aggr.at[idv.at[k]], SS[j]).wait()

    def compute(rows_, eav):
        def quad(q, inner_carry):
            av = eav[q, :]
            for je in range(4):
                i = 4 * q + je
                a = [av[4 * je + kk] for kk in range(ED)]
                for j in range(H // VL):
                    sl = pl.ds(VL * j, VL)
                    e = bb[j]
                    for kk in range(ED):
                        e = e + a[kk] * wk[kk][j]
                    rows_[i, sl] = jnp.maximum(rows_[i, sl] + e, 0.0)
            return inner_carry

        lax.fori_loop(0, C // 4, quad, 0)

    start_fetch(0, 0)
    start_fetch(1, 1)
    start_fetch(2, 2)
    wait_fetch(0)
    start_gather(0)
    wait_fetch(1)
    start_gather(1)

    def triple(t, carry):
        for i in range(3):
            k = 3 * t + i
            j2 = (i + 2) % 3
            wait_gather(i)
            compute(ROWS[i], EAV[i])
            start_scatter(k, i)

            @pl.when(k + 3 < NCH)
            def _():
                start_fetch(k + 3, i)

            @pl.when(k >= 1)
            def _():
                wait_scatter(k - 1, j2)

            wait_fetch(j2)
            start_gather(j2)
        return carry

    lax.fori_loop(0, NCH // 3, triple, 0)
    wait_gather(0)
    compute(ROWS[0], EAV[0])
    start_scatter(NCH - 2, 0)
    wait_gather(1)
    compute(ROWS[1], EAV[1])
    start_scatter(NCH - 1, 1)
    wait_scatter(NCH - 3, 2)
    wait_scatter(NCH - 2, 0)
    wait_scatter(NCH - 1, 1)
    plsc.subcore_barrier()
    for t in range((N // C + NS - 1) // NS):
        i = s + NS * t
        @pl.when(i < N // C)
        def _():
            pltpu.sync_copy(aggr.at[pl.ds(i * C, C), :],
                            out_hbm.at[pl.ds(c * N + i * C, C), :])


@functools.cache
def _pool_kernel():
    return pl.kernel(
        _pool_body,
        out_type=[jax.ShapeDtypeStruct((NW, 2, L * H), jnp.float32),
                  jax.ShapeDtypeStruct((NW, 2, L * H), jnp.float32)],
        mesh=_mesh(),
        scratch_types=[
            pltpu.VMEM((N,), jnp.int32),
            pltpu.VMEM((PCP, H), jnp.float32),
            pltpu.VMEM((PCP, H), jnp.float32),
            pltpu.VMEM((PCP, H), jnp.float32),
            pltpu.VMEM((L * H,), jnp.float32),
            pltpu.VMEM((L * H,), jnp.float32),
            pltpu.VMEM((2, L * H), jnp.float32),
            pltpu.VMEM((2, L * H), jnp.float32),
        ],
        compiler_params=_SC_PARAMS,
    )


def _pool_body(h1_hbm, h2_hbm, h3_hbm, b_hbm, mean_hbm, max_hbm,
               bv, r1, r2, r3, sacc, macc, mrow, xrow):
    c = lax.axis_index("c")
    s = lax.axis_index("s")
    w = c * NS + s
    pltpu.sync_copy(b_hbm, bv)
    hs = (h1_hbm, h2_hbm, h3_hbm)
    rs = (r1, r2, r3)

    for gl in range(2):
        g = 2 * w + gl

        def cnt(i, carry):
            sg, eg = carry
            v = bv[pl.ds(VL * i, VL)]
            sg = sg + jnp.sum((v < g).astype(jnp.int32))
            eg = eg + jnp.sum((v <= g).astype(jnp.int32))
            return sg, eg

        sg, eg = lax.fori_loop(0, N // VL, cnt,
                               (jnp.int32(0), jnp.int32(0)))

        for j in range(L * H // VL):
            sacc[pl.ds(VL * j, VL)] = jnp.zeros((VL,), jnp.float32)
            macc[pl.ds(VL * j, VL)] = jnp.full((VL,), -jnp.inf, jnp.float32)

        k0 = sg // PCP
        k1 = (eg + PCP - 1) // PCP

        def chunk(k, carry):
            rbase = k * PCP
            for t in range(L):
                pltpu.sync_copy(hs[t].at[pl.ds(rbase, PCP), :], rs[t])

            def row(i, inner_carry):
                ridx = jnp.full((VL,), rbase + i, jnp.int32)
                valid = (ridx >= sg) & (ridx < eg)
                for t in range(L):
                    for j in range(H // VL):
                        v = rs[t][i, pl.ds(VL * j, VL)]
                        asl = pl.ds(t * H + VL * j, VL)
                        sacc[asl] = sacc[asl] + jnp.where(valid, v, 0.0)
                        macc[asl] = jnp.maximum(
                            macc[asl], jnp.where(valid, v, -jnp.inf))
                return inner_carry

            lax.fori_loop(0, PCP, row, 0)
            return carry

        lax.fori_loop(k0, k1, chunk, 0)
        denom = jnp.maximum((eg - sg).astype(jnp.float32), 1.0)
        for j in range(L * H // VL):
            sl = pl.ds(VL * j, VL)
            mrow[gl, sl] = sacc[sl] / denom
            xrow[gl, sl] = macc[sl]

    pltpu.sync_copy(mrow, mean_hbm.at[w])
    pltpu.sync_copy(xrow, max_hbm.at[w])


def kernel(x, edge_index, edge_attr, batch, W_in, b_in, We, be, W1, b1,
           gm, bm, W2, b2, go, bo, eps, Wh1, bh1, gh, bh, Wh2, bh2):
    src = edge_index[0].astype(jnp.int32)
    dst = edge_index[1].astype(jnp.int32)
    batch32 = batch.astype(jnp.int32)

    s1 = gm / jnp.sqrt(1.0 + BN_EPS)
    w1f = W1 * s1[:, None, :]
    b1f = b1 * s1 + bm
    s2 = go / jnp.sqrt(1.0 + BN_EPS)
    w2f = W2 * s2[:, None, :]
    b2f = b2 * s2 + bo
    sh = gh / jnp.sqrt(1.0 + BN_EPS)
    wh1f = Wh1 * sh[None, :]
    bh1f = bh1 * sh + bh

    h = _linear(x, W_in, b_in)
    src3 = src.reshape(NW * NCH, C)
    dst3 = dst.reshape(NW, NCH, C)
    ea4 = edge_attr.astype(jnp.float32).reshape(NW * NCH, C // 4, 4 * ED)

    outs = []
    for l in range(L):
        parts = _edge_pass_kernel()(src3, dst3, ea4, We[l], be[l], h)
        scale_row = jnp.broadcast_to(
            (1.0 + eps[l]).astype(jnp.float32)[None, None], (1, H))
        h = _mlp(h, parts.reshape(NC, N, H), scale_row,
                 w1f[l], b1f[l], w2f[l], b2f[l])
        outs.append(h)

    means, maxes = _pool_kernel()(outs[0], outs[1], outs[2], batch32)
    o = _head(means.reshape(G, L * H), maxes.reshape(G, L * H),
              wh1f[:L * H], wh1f[L * H:], bh1f, Wh2, bh2)
    return o[:, 0]
